# baseline (speedup 1.0000x reference)
import functools
import jax
import jax.numpy as jnp
from jax import lax
from jax.experimental import pallas as pl
from jax.experimental.pallas import tpu as pltpu
from jax.experimental.pallas import tpu_sc as plsc

_NW = 32
_CHUNK = 16384


def _make_sc(total):
    per_w = total // _NW
    n_chunks = per_w // _CHUNK
    mesh = plsc.VectorSubcoreMesh(core_axis_name="c", subcore_axis_name="s")

    @functools.partial(
        pl.kernel,
        mesh=mesh,
        out_type=jax.ShapeDtypeStruct((total,), jnp.float32),
        scratch_types=[
            pltpu.VMEM((_CHUNK,), jnp.float32),
            pltpu.VMEM((_CHUNK,), jnp.float32),
        ],
    )
    def k(x_hbm, out_hbm, buf_in, buf_out):
        wid = lax.axis_index("s") * 2 + lax.axis_index("c")
        base = wid * per_w
        lane = lax.iota(jnp.int32, 16)
        grp = lane & ~3
        p1 = (grp | ((lane + 1) & 3))[:, None]
        p2 = (grp | ((lane + 2) & 3))[:, None]
        p3 = (grp | ((lane + 3) & 3))[:, None]
        dnums = lax.GatherDimensionNumbers(
            offset_dims=(), collapsed_slice_dims=(0,), start_index_map=(0,)
        )

        def _perm(v, p):
            return lax.gather(
                v, p, dnums, slice_sizes=(1,),
                mode=lax.GatherScatterMode.PROMISE_IN_BOUNDS,
            )

        def chunk_body(ci, carry):
            off = base + ci * _CHUNK
            pltpu.sync_copy(x_hbm.at[pl.ds(off, _CHUNK)], buf_in)

            def vec_body(j, c2):
                b = j * 16
                v = buf_in[pl.ds(b, 16)]
                y1 = _perm(v, p1)
                y2 = _perm(v, p2)
                y3 = _perm(v, p3)
                mx1 = jnp.maximum(v, y1)
                mn1 = jnp.minimum(v, y1)
                mx2 = jnp.maximum(y2, y3)
                mn2 = jnp.minimum(y2, y3)
                second = jnp.maximum(
                    jnp.minimum(mx1, mx2), jnp.maximum(mn1, mn2)
                )
                buf_out[pl.ds(b, 16)] = jnp.where(v >= second, v, 0.0)
                return c2

            lax.fori_loop(0, _CHUNK // 16, vec_body, 0)
            pltpu.sync_copy(buf_out, out_hbm.at[pl.ds(off, _CHUNK)])
            return carry

        lax.fori_loop(0, n_chunks, chunk_body, 0)

    return k


def kernel(input):
    n, d = input.shape
    flat = input.reshape(n * d)
    out = _make_sc(n * d)(flat)
    return out.reshape(n, d)


# speedup vs baseline: 1.1402x; 1.1402x over previous
import functools
import jax
import jax.numpy as jnp
from jax import lax
from jax.experimental import pallas as pl
from jax.experimental.pallas import tpu as pltpu
from jax.experimental.pallas import tpu_sc as plsc

_NW = 32
_CHUNK = 16384
_UNROLL = 8


def _make_sc(total):
    per_w = total // _NW
    n_chunks = per_w // _CHUNK
    mesh = plsc.VectorSubcoreMesh(core_axis_name="c", subcore_axis_name="s")

    @functools.partial(
        pl.kernel,
        mesh=mesh,
        out_type=jax.ShapeDtypeStruct((total,), jnp.float32),
        scratch_types=[
            pltpu.VMEM((_CHUNK,), jnp.float32),
            pltpu.VMEM((_CHUNK,), jnp.float32),
        ],
    )
    def k(x_hbm, out_hbm, buf_in, buf_out):
        wid = lax.axis_index("s") * 2 + lax.axis_index("c")
        base = wid * per_w
        lane = lax.iota(jnp.int32, 16)
        grp = lane & ~3
        p1 = (grp | ((lane + 1) & 3))[:, None]
        p2 = (grp | ((lane + 2) & 3))[:, None]
        p3 = (grp | ((lane + 3) & 3))[:, None]
        dnums = lax.GatherDimensionNumbers(
            offset_dims=(), collapsed_slice_dims=(0,), start_index_map=(0,)
        )

        def _perm(v, p):
            return lax.gather(
                v, p, dnums, slice_sizes=(1,),
                mode=lax.GatherScatterMode.PROMISE_IN_BOUNDS,
            )

        def chunk_body(ci, carry):
            off = base + ci * _CHUNK
            pltpu.sync_copy(x_hbm.at[pl.ds(off, _CHUNK)], buf_in)

            def vec_body(j, c2):
                for u in range(_UNROLL):
                    b = j * (16 * _UNROLL) + u * 16
                    v = buf_in[pl.ds(b, 16)]
                    y1 = _perm(v, p1)
                    y2 = _perm(v, p2)
                    y3 = _perm(v, p3)
                    mx1 = jnp.maximum(v, y1)
                    mn1 = jnp.minimum(v, y1)
                    mx2 = jnp.maximum(y2, y3)
                    mn2 = jnp.minimum(y2, y3)
                    second = jnp.maximum(
                        jnp.minimum(mx1, mx2), jnp.maximum(mn1, mn2)
                    )
                    buf_out[pl.ds(b, 16)] = jnp.where(v >= second, v, 0.0)
                return c2

            lax.fori_loop(0, _CHUNK // (16 * _UNROLL), vec_body, 0)
            pltpu.sync_copy(buf_out, out_hbm.at[pl.ds(off, _CHUNK)])
            return carry

        lax.fori_loop(0, n_chunks, chunk_body, 0)

    return k


def kernel(input):
    n, d = input.shape
    flat = input.reshape(n * d)
    out = _make_sc(n * d)(flat)
    return out.reshape(n, d)


# speedup vs baseline: 1.3681x; 1.1998x over previous
import functools
import jax
import jax.numpy as jnp
from jax import lax
from jax.experimental import pallas as pl
from jax.experimental.pallas import tpu as pltpu
from jax.experimental.pallas import tpu_sc as plsc

_BLOCK_ROWS = 256
_TC_ROWS = 5376
_NW = 32
_CHUNK = 16384
_UNROLL = 8


def _tc_body(x_ref, o_ref):
    x = x_ref[...]
    r, d = x.shape
    p = jax.lax.broadcasted_iota(jnp.int32, (r, d), 1) & 3
    right1 = pltpu.roll(x, d - 1, 1)
    left1 = pltpu.roll(x, 1, 1)
    s1 = jnp.where((p & 1) == 0, right1, left1)
    mx = jnp.maximum(x, s1)
    mn = jnp.minimum(x, s1)
    lo = p < 2
    mx_sw = jnp.where(lo, pltpu.roll(mx, d - 2, 1), pltpu.roll(mx, 2, 1))
    mn_sw = jnp.where(lo, pltpu.roll(mn, d - 2, 1), pltpu.roll(mn, 2, 1))
    second = jnp.maximum(jnp.minimum(mx, mx_sw), jnp.maximum(mn, mn_sw))
    o_ref[...] = jnp.where(x >= second, x, jnp.zeros_like(x))


def _tc_kernel(x):
    n, d = x.shape
    grid = n // _BLOCK_ROWS
    return pl.pallas_call(
        _tc_body,
        grid=(grid,),
        in_specs=[pl.BlockSpec((_BLOCK_ROWS, d), lambda i: (i, 0))],
        out_specs=pl.BlockSpec((_BLOCK_ROWS, d), lambda i: (i, 0)),
        out_shape=jax.ShapeDtypeStruct((n, d), x.dtype),
        compiler_params=pltpu.CompilerParams(
            dimension_semantics=("arbitrary",),
        ),
    )(x)


def _make_sc(total):
    per_w = total // _NW
    n_chunks = per_w // _CHUNK
    mesh = plsc.VectorSubcoreMesh(core_axis_name="c", subcore_axis_name="s")

    @functools.partial(
        pl.kernel,
        mesh=mesh,
        out_type=jax.ShapeDtypeStruct((total,), jnp.float32),
        scratch_types=[
            pltpu.VMEM((_CHUNK,), jnp.float32),
            pltpu.VMEM((_CHUNK,), jnp.float32),
        ],
    )
    def k(x_hbm, out_hbm, buf_in, buf_out):
        wid = lax.axis_index("s") * 2 + lax.axis_index("c")
        base = wid * per_w
        lane = lax.iota(jnp.int32, 16)
        grp = lane & ~3
        p1 = (grp | ((lane + 1) & 3))[:, None]
        p2 = (grp | ((lane + 2) & 3))[:, None]
        p3 = (grp | ((lane + 3) & 3))[:, None]
        dnums = lax.GatherDimensionNumbers(
            offset_dims=(), collapsed_slice_dims=(0,), start_index_map=(0,)
        )

        def _perm(v, p):
            return lax.gather(
                v, p, dnums, slice_sizes=(1,),
                mode=lax.GatherScatterMode.PROMISE_IN_BOUNDS,
            )

        def chunk_body(ci, carry):
            off = base + ci * _CHUNK
            pltpu.sync_copy(x_hbm.at[pl.ds(off, _CHUNK)], buf_in)

            def vec_body(j, c2):
                for u in range(_UNROLL):
                    b = j * (16 * _UNROLL) + u * 16
                    v = buf_in[pl.ds(b, 16)]
                    y1 = _perm(v, p1)
                    y2 = _perm(v, p2)
                    y3 = _perm(v, p3)
                    mx1 = jnp.maximum(v, y1)
                    mn1 = jnp.minimum(v, y1)
                    mx2 = jnp.maximum(y2, y3)
                    mn2 = jnp.minimum(y2, y3)
                    second = jnp.maximum(
                        jnp.minimum(mx1, mx2), jnp.maximum(mn1, mn2)
                    )
                    buf_out[pl.ds(b, 16)] = jnp.where(v >= second, v, 0.0)
                return c2

            lax.fori_loop(0, _CHUNK // (16 * _UNROLL), vec_body, 0)
            pltpu.sync_copy(buf_out, out_hbm.at[pl.ds(off, _CHUNK)])
            return carry

        lax.fori_loop(0, n_chunks, chunk_body, 0)

    return k


def kernel(input):
    n, d = input.shape
    top = input[:_TC_ROWS]
    bot = input[_TC_ROWS:].reshape((n - _TC_ROWS) * d)
    out_sc = _make_sc(bot.shape[0])(bot)
    out_tc = _tc_kernel(top)
    return jnp.concatenate([out_tc, out_sc.reshape(n - _TC_ROWS, d)], axis=0)


# speedup vs baseline: 2.4684x; 1.8042x over previous
import jax
import jax.numpy as jnp
from jax.experimental import pallas as pl
from jax.experimental.pallas import tpu as pltpu

_BLOCK_ROWS = 128


def _body(x_ref, o_ref):
    x = x_ref[...]
    r, d = x.shape
    p = jax.lax.broadcasted_iota(jnp.int32, (r, d), 1) & 3
    right1 = pltpu.roll(x, d - 1, 1)
    left1 = pltpu.roll(x, 1, 1)
    s1 = jnp.where((p & 1) == 0, right1, left1)
    mx = jnp.maximum(x, s1)
    mn = jnp.minimum(x, s1)
    lo = p < 2
    mx_sw = jnp.where(lo, pltpu.roll(mx, d - 2, 1), pltpu.roll(mx, 2, 1))
    mn_sw = jnp.where(lo, pltpu.roll(mn, d - 2, 1), pltpu.roll(mn, 2, 1))
    second = jnp.maximum(jnp.minimum(mx, mx_sw), jnp.maximum(mn, mn_sw))
    o_ref[...] = jnp.where(x >= second, x, jnp.zeros_like(x))


def kernel(input):
    n, d = input.shape
    grid = n // _BLOCK_ROWS
    return pl.pallas_call(
        _body,
        grid=(grid,),
        in_specs=[pl.BlockSpec((_BLOCK_ROWS, d), lambda i: (i, 0))],
        out_specs=pl.BlockSpec((_BLOCK_ROWS, d), lambda i: (i, 0)),
        out_shape=jax.ShapeDtypeStruct((n, d), input.dtype),
        compiler_params=pltpu.CompilerParams(
            dimension_semantics=("arbitrary",),
        ),
    )(input)
